# baseline (speedup 1.0000x reference)
import functools

import jax
import jax.numpy as jnp
from jax import lax
from jax.experimental import pallas as pl
from jax.experimental.pallas import tpu as pltpu
from jax.experimental.pallas import tpu_sc as plsc

B = 100
NPG = 100
D = 128
N = B * NPG
E = 320000
K1, K2, K3 = 50, 25, 13


_NC = 2
_NS = 16
_NW = _NC * _NS
_EPW = E // _NW
_BK = 128
_NB = _EPW // _BK
_TL = _EPW - _NB * _BK
_CH = 400
_NCH = N // _CH


def _sc_segsum_body(h_hbm, src_hbm, dst_hbm, zeros_hbm, out_hbm,
                    src_all, d0, d1, rows0, rows1, dt, rowst, acc,
                    gsem0, gsem1, dsem0, dsem1, tsem):
    c = lax.axis_index("c")
    s = lax.axis_index("s")
    wid = s * _NC + c
    base = wid * _EPW

    for j in range(2):
        cid = s + j * _NS

        @pl.when(cid < _NCH)
        def _():
            pltpu.sync_copy(zeros_hbm, acc.at[pl.ds(cid * _CH, _CH)])

    pltpu.sync_copy(src_hbm.at[pl.ds(base, _EPW)], src_all)
    plsc.subcore_barrier()

    pltpu.async_copy(h_hbm.at[src_all.at[pl.ds(0, _BK)]], rows0, gsem0)
    pltpu.async_copy(h_hbm.at[src_all.at[pl.ds(_BK, _BK)]], rows1, gsem1)
    pltpu.async_copy(dst_hbm.at[pl.ds(base, _BK)], d0, dsem0)
    pltpu.async_copy(dst_hbm.at[pl.ds(base + _BK, _BK)], d1, dsem1)

    def slot(bid, dv, rows, gsem, dsem):
        pltpu.make_async_copy(dst_hbm.at[pl.ds(base, _BK)], dv, dsem).wait()
        pltpu.make_async_copy(h_hbm.at[src_all.at[pl.ds(0, _BK)]],
                              rows, gsem).wait()
        pltpu.sync_copy(rows, acc.at[dv], add=True)

        @pl.when(bid + 2 < _NB)
        def _():
            nxt = (bid + 2) * _BK
            pltpu.async_copy(h_hbm.at[src_all.at[pl.ds(nxt, _BK)]],
                             rows, gsem)
            pltpu.async_copy(dst_hbm.at[pl.ds(base + nxt, _BK)], dv, dsem)

    def body(jj, carry):
        slot(2 * jj, d0, rows0, gsem0, dsem0)
        slot(2 * jj + 1, d1, rows1, gsem1, dsem1)
        return carry

    lax.fori_loop(0, _NB // 2, body, 0)

    pltpu.async_copy(h_hbm.at[src_all.at[pl.ds(_NB * _BK, _TL)]],
                     rowst, tsem).wait()
    pltpu.sync_copy(dst_hbm.at[pl.ds(base + _NB * _BK, _TL)], dt)
    pltpu.sync_copy(rowst, acc.at[dt], add=True)

    plsc.subcore_barrier()
    for j in range(2):
        cid = s + j * _NS

        @pl.when(cid < _NCH)
        def _():
            pltpu.sync_copy(acc.at[pl.ds(cid * _CH, _CH)],
                            out_hbm.at[c, pl.ds(cid * _CH, _CH)])


@functools.cache
def _sc_segsum_fn():
    return pl.kernel(
        _sc_segsum_body,
        out_type=jax.ShapeDtypeStruct((_NC, N, D), jnp.float32),
        mesh=plsc.VectorSubcoreMesh(core_axis_name="c", subcore_axis_name="s"),
        scratch_types=[
            pltpu.VMEM((_EPW,), jnp.int32),
            pltpu.VMEM((_BK,), jnp.int32),
            pltpu.VMEM((_BK,), jnp.int32),
            pltpu.VMEM((_BK, D), jnp.float32),
            pltpu.VMEM((_BK, D), jnp.float32),
            pltpu.VMEM((_TL,), jnp.int32),
            pltpu.VMEM((_TL, D), jnp.float32),
            pltpu.VMEM_SHARED((N, D), jnp.float32),
            pltpu.SemaphoreType.DMA,
            pltpu.SemaphoreType.DMA,
            pltpu.SemaphoreType.DMA,
            pltpu.SemaphoreType.DMA,
            pltpu.SemaphoreType.DMA,
        ],
    )


def _sc_segsum(h, src, dst, zeros):
    return _sc_segsum_fn()(h, src, dst, zeros)


_GB = 10
_RB = _GB * NPG
_NEG = -3e38


def _graph_pool(part_ref, hp_ref, wrel_ref, brel_ref, wroot_ref, q_ref,
                pn_ref, og_ref, g, k):
    r0 = pl.ds(g * NPG, NPG)
    aggr = part_ref[0, r0, :] + part_ref[1, r0, :]
    t = (jnp.dot(aggr, wrel_ref[...], preferred_element_type=jnp.float32)
         + brel_ref[...]
         + jnp.dot(hp_ref[r0, :], wroot_ref[...],
                   preferred_element_type=jnp.float32))
    h = jnp.maximum(t, 0.0)
    u = jnp.dot(h, q_ref[...], preferred_element_type=jnp.float32)
    score = jnp.tanh(u / pn_ref[0, 0])[:, 0]
    ordk = og_ref[0, g, :]
    alive = ordk < NPG
    s = jnp.where(alive, score, _NEG)
    a = s[:, None]
    b = s[None, :]
    oi = ordk[:, None]
    oj = ordk[None, :]
    better = (b > a) | ((b == a) & (oj < oi))
    rank = jnp.sum(better.astype(jnp.float32), axis=1).astype(jnp.int32)
    keep = (rank < k) & alive
    m = jnp.where(keep, score, 0.0)
    hm = h * m[:, None]
    mx = jnp.max(jnp.where(keep[:, None], hm, _NEG), axis=0)
    mn = jnp.sum(hm, axis=0) / k
    return hm, keep, rank, mx, mn


def _tcs_body(part_ref, hp_ref, wrel_ref, brel_ref, wroot_ref, q_ref,
              pn_ref, og_ref, hm_ref, on_ref, r_ref, *, k):
    for g in range(_GB):
        hm, keep, rank, mx, mn = _graph_pool(
            part_ref, hp_ref, wrel_ref, brel_ref, wroot_ref, q_ref,
            pn_ref, og_ref, g, k)
        hm_ref[pl.ds(g * NPG, NPG), :] = hm
        on_ref[0, g, :] = jnp.where(keep, rank, NPG)
        r_ref[0, g, pl.ds(0, D)] = mx
        r_ref[0, g, pl.ds(D, D)] = mn


def _tcs(part, hprev, wrel, brel, wroot, q, pn, og, k):
    return pl.pallas_call(
        functools.partial(_tcs_body, k=k),
        grid=(N // _RB,),
        in_specs=[
            pl.BlockSpec((_NC, _RB, D), lambda i: (0, i, 0)),
            pl.BlockSpec((_RB, D), lambda i: (i, 0)),
            pl.BlockSpec((D, D), lambda i: (0, 0)),
            pl.BlockSpec((1, D), lambda i: (0, 0)),
            pl.BlockSpec((D, D), lambda i: (0, 0)),
            pl.BlockSpec((D, 1), lambda i: (0, 0)),
            pl.BlockSpec(memory_space=pltpu.SMEM),
            pl.BlockSpec((1, _GB, NPG), lambda i: (i, 0, 0)),
        ],
        out_specs=[
            pl.BlockSpec((_RB, D), lambda i: (i, 0)),
            pl.BlockSpec((1, _GB, NPG), lambda i: (i, 0, 0)),
            pl.BlockSpec((1, _GB, 2 * D), lambda i: (i, 0, 0)),
        ],
        out_shape=[
            jax.ShapeDtypeStruct((N, D), jnp.float32),
            jax.ShapeDtypeStruct((B // _GB, _GB, NPG), jnp.int32),
            jax.ShapeDtypeStruct((B // _GB, _GB, 2 * D), jnp.float32),
        ],
    )(part, hprev, wrel, brel, wroot, q, pn, og)


def _tcf_body(part_ref, hp_ref, wrel_ref, brel_ref, wroot_ref, q_ref,
              pn_ref, og_ref, r1_ref, r2_ref,
              w1_ref, b1_ref, w2_ref, b2_ref, w3_ref, b3_ref, out_ref, *, k):
    for g in range(_GB):
        _, _, _, mx, mn = _graph_pool(
            part_ref, hp_ref, wrel_ref, brel_ref, wroot_ref, q_ref,
            pn_ref, og_ref, g, k)
        rg = pl.ds(g, 1)
        z = (r1_ref[0, rg, :] + r2_ref[0, rg, :]
             + jnp.concatenate([mx[None, :], mn[None, :]], axis=1))
        z = jnp.maximum(jnp.dot(z, w1_ref[...],
                                preferred_element_type=jnp.float32)
                        + b1_ref[...], 0.0)
        z = jnp.maximum(jnp.dot(z, w2_ref[...],
                                preferred_element_type=jnp.float32)
                        + b2_ref[...], 0.0)
        logits = jnp.dot(z, w3_ref[...],
                         preferred_element_type=jnp.float32) + b3_ref[...]
        mxl = jnp.max(logits, axis=1, keepdims=True)
        lse = jnp.log(jnp.sum(jnp.exp(logits - mxl), axis=1, keepdims=True))
        out_ref[0, rg, :] = logits - mxl - lse


def _tcf(part, hprev, wrel, brel, wroot, q, pn, og, r1, r2,
         w1, b1, w2, b2, w3, b3, k):
    return pl.pallas_call(
        functools.partial(_tcf_body, k=k),
        grid=(N // _RB,),
        in_specs=[
            pl.BlockSpec((_NC, _RB, D), lambda i: (0, i, 0)),
            pl.BlockSpec((_RB, D), lambda i: (i, 0)),
            pl.BlockSpec((D, D), lambda i: (0, 0)),
            pl.BlockSpec((1, D), lambda i: (0, 0)),
            pl.BlockSpec((D, D), lambda i: (0, 0)),
            pl.BlockSpec((D, 1), lambda i: (0, 0)),
            pl.BlockSpec(memory_space=pltpu.SMEM),
            pl.BlockSpec((1, _GB, NPG), lambda i: (i, 0, 0)),
            pl.BlockSpec((1, _GB, 2 * D), lambda i: (i, 0, 0)),
            pl.BlockSpec((1, _GB, 2 * D), lambda i: (i, 0, 0)),
            pl.BlockSpec((2 * D, D), lambda i: (0, 0)),
            pl.BlockSpec((1, D), lambda i: (0, 0)),
            pl.BlockSpec((D, 64), lambda i: (0, 0)),
            pl.BlockSpec((1, 64), lambda i: (0, 0)),
            pl.BlockSpec((64, 10), lambda i: (0, 0)),
            pl.BlockSpec((1, 10), lambda i: (0, 0)),
        ],
        out_specs=pl.BlockSpec((1, _GB, 10), lambda i: (i, 0, 0)),
        out_shape=jax.ShapeDtypeStruct((B // _GB, _GB, 10), jnp.float32),
    )(part, hprev, wrel, brel, wroot, q, pn, og, r1, r2,
      w1, b1, w2, b2, w3, b3)


def kernel(x, edge_index, batch, conv1_Wrel, conv1_brel, conv1_Wroot, p1,
           conv2_Wrel, conv2_brel, conv2_Wroot, p2,
           conv3_Wrel, conv3_brel, conv3_Wroot, p3,
           lin1_W, lin1_b, lin2_W, lin2_b, lin3_W, lin3_b):
    src = edge_index[0]
    dst = edge_index[1]
    zeros = jnp.zeros((_CH, D), jnp.float32)
    ord0 = jnp.tile(jnp.arange(NPG, dtype=jnp.int32), B).reshape(
        B // _GB, _GB, NPG)

    def pnorm(p):
        return (jnp.linalg.norm(p) + 1e-16).reshape(1, 1)

    part = _sc_segsum(x, src, dst, zeros)
    hm, ord1, r1 = _tcs(part, x, conv1_Wrel, conv1_brel.reshape(1, D),
                        conv1_Wroot, p1.reshape(D, 1), pnorm(p1), ord0, K1)

    part = _sc_segsum(hm, src, dst, zeros)
    hm, ord2, r2 = _tcs(part, hm, conv2_Wrel, conv2_brel.reshape(1, D),
                        conv2_Wroot, p2.reshape(D, 1), pnorm(p2), ord1, K2)

    part = _sc_segsum(hm, src, dst, zeros)
    out3 = _tcf(part, hm, conv3_Wrel, conv3_brel.reshape(1, D), conv3_Wroot,
                p3.reshape(D, 1), pnorm(p3), ord2, r1, r2,
                lin1_W, lin1_b.reshape(1, D), lin2_W, lin2_b.reshape(1, 64),
                lin3_W, lin3_b.reshape(1, 10), K3)
    return out3.reshape(B, 10)


# speedup vs baseline: 1.0768x; 1.0768x over previous
import functools

import jax
import jax.numpy as jnp
from jax import lax
from jax.experimental import pallas as pl
from jax.experimental.pallas import tpu as pltpu
from jax.experimental.pallas import tpu_sc as plsc

B = 100
NPG = 100
D = 128
N = B * NPG
E = 320000
K1, K2, K3 = 50, 25, 13


_NC = 2
_NS = 16
_NW = _NC * _NS
_EPW = E // _NW
_BK = 128
_NB = _EPW // _BK
_TL = _EPW - _NB * _BK
_CH = 400
_NCH = N // _CH


def _sc_segsum_body(h_hbm, src_hbm, dst_hbm, zeros_hbm, out_hbm,
                    src_all, d0, d1, rows0, rows1, dt, rowst, acc,
                    gsem0, gsem1, dsem0, dsem1, tsem):
    c = lax.axis_index("c")
    s = lax.axis_index("s")
    wid = s * _NC + c
    base = wid * _EPW

    for j in range(2):
        cid = s + j * _NS

        @pl.when(cid < _NCH)
        def _():
            pltpu.sync_copy(zeros_hbm, acc.at[pl.ds(cid * _CH, _CH)])

    pltpu.sync_copy(src_hbm.at[pl.ds(base, _EPW)], src_all)
    plsc.subcore_barrier()

    pltpu.async_copy(h_hbm.at[src_all.at[pl.ds(0, _BK)]], rows0, gsem0)
    pltpu.async_copy(h_hbm.at[src_all.at[pl.ds(_BK, _BK)]], rows1, gsem1)
    pltpu.async_copy(dst_hbm.at[pl.ds(base, _BK)], d0, dsem0)
    pltpu.async_copy(dst_hbm.at[pl.ds(base + _BK, _BK)], d1, dsem1)

    def slot(bid, dv, rows, gsem, dsem):
        pltpu.make_async_copy(dst_hbm.at[pl.ds(base, _BK)], dv, dsem).wait()
        pltpu.make_async_copy(h_hbm.at[src_all.at[pl.ds(0, _BK)]],
                              rows, gsem).wait()
        pltpu.sync_copy(rows, acc.at[dv], add=True)

        @pl.when(bid + 2 < _NB)
        def _():
            nxt = (bid + 2) * _BK
            pltpu.async_copy(h_hbm.at[src_all.at[pl.ds(nxt, _BK)]],
                             rows, gsem)
            pltpu.async_copy(dst_hbm.at[pl.ds(base + nxt, _BK)], dv, dsem)

    def body(jj, carry):
        slot(2 * jj, d0, rows0, gsem0, dsem0)
        slot(2 * jj + 1, d1, rows1, gsem1, dsem1)
        return carry

    lax.fori_loop(0, _NB // 2, body, 0)

    pltpu.async_copy(h_hbm.at[src_all.at[pl.ds(_NB * _BK, _TL)]],
                     rowst, tsem).wait()
    pltpu.sync_copy(dst_hbm.at[pl.ds(base + _NB * _BK, _TL)], dt)
    pltpu.sync_copy(rowst, acc.at[dt], add=True)

    plsc.subcore_barrier()
    for j in range(2):
        cid = s + j * _NS

        @pl.when(cid < _NCH)
        def _():
            pltpu.sync_copy(acc.at[pl.ds(cid * _CH, _CH)],
                            out_hbm.at[c, pl.ds(cid * _CH, _CH)])


@functools.cache
def _sc_segsum_fn():
    return pl.kernel(
        _sc_segsum_body,
        out_type=jax.ShapeDtypeStruct((_NC, N, D), jnp.float32),
        mesh=plsc.VectorSubcoreMesh(core_axis_name="c", subcore_axis_name="s"),
        scratch_types=[
            pltpu.VMEM((_EPW,), jnp.int32),
            pltpu.VMEM((_BK,), jnp.int32),
            pltpu.VMEM((_BK,), jnp.int32),
            pltpu.VMEM((_BK, D), jnp.float32),
            pltpu.VMEM((_BK, D), jnp.float32),
            pltpu.VMEM((_TL,), jnp.int32),
            pltpu.VMEM((_TL, D), jnp.float32),
            pltpu.VMEM_SHARED((N, D), jnp.float32),
            pltpu.SemaphoreType.DMA,
            pltpu.SemaphoreType.DMA,
            pltpu.SemaphoreType.DMA,
            pltpu.SemaphoreType.DMA,
            pltpu.SemaphoreType.DMA,
        ],
    )


def _sc_segsum(h, src, dst, zeros):
    return _sc_segsum_fn()(h, src, dst, zeros)


_GB = 10
_RB = _GB * NPG
_NEG = -3e38


def _block_conv(part_ref, hp_ref, wrel_ref, brel_ref, wroot_ref, q_ref,
                pn_ref):
    aggr = part_ref[0] + part_ref[1]
    t = (jnp.dot(aggr, wrel_ref[...], preferred_element_type=jnp.float32)
         + brel_ref[...]
         + jnp.dot(hp_ref[...], wroot_ref[...],
                   preferred_element_type=jnp.float32))
    h = jnp.maximum(t, 0.0)
    u = jnp.dot(h, q_ref[...], preferred_element_type=jnp.float32)
    score = jnp.tanh(u / pn_ref[0, 0])
    return h, score


def _graph_pool(h, score, og_ref, g, k):
    sc = score[g * NPG:(g + 1) * NPG, 0]
    hg = h[g * NPG:(g + 1) * NPG, :]
    ordk = og_ref[0, g, :]
    alive = ordk < NPG
    s = jnp.where(alive, sc, _NEG)
    a = s[:, None]
    b = s[None, :]
    oi = ordk[:, None]
    oj = ordk[None, :]
    better = (b > a) | ((b == a) & (oj < oi))
    rank = jnp.sum(better.astype(jnp.float32), axis=1).astype(jnp.int32)
    keep = (rank < k) & alive
    m = jnp.where(keep, sc, 0.0)
    hm = hg * m[:, None]
    mx = jnp.max(jnp.where(keep[:, None], hm, _NEG), axis=0)
    mn = jnp.sum(hm, axis=0) / k
    return keep, rank, m, mx, mn


def _tcs_body(part_ref, hp_ref, wrel_ref, brel_ref, wroot_ref, q_ref,
              pn_ref, og_ref, hm_ref, on_ref, r_ref, *, k):
    h, score = _block_conv(part_ref, hp_ref, wrel_ref, brel_ref, wroot_ref,
                           q_ref, pn_ref)
    ms = []
    for g in range(_GB):
        keep, rank, m, mx, mn = _graph_pool(h, score, og_ref, g, k)
        ms.append(m[:, None])
        on_ref[0, g, :] = jnp.where(keep, rank, NPG)
        r_ref[0, g, pl.ds(0, D)] = mx
        r_ref[0, g, pl.ds(D, D)] = mn
    hm_ref[...] = h * jnp.concatenate(ms, axis=0)


def _tcs(part, hprev, wrel, brel, wroot, q, pn, og, k):
    return pl.pallas_call(
        functools.partial(_tcs_body, k=k),
        grid=(N // _RB,),
        in_specs=[
            pl.BlockSpec((_NC, _RB, D), lambda i: (0, i, 0)),
            pl.BlockSpec((_RB, D), lambda i: (i, 0)),
            pl.BlockSpec((D, D), lambda i: (0, 0)),
            pl.BlockSpec((1, D), lambda i: (0, 0)),
            pl.BlockSpec((D, D), lambda i: (0, 0)),
            pl.BlockSpec((D, 1), lambda i: (0, 0)),
            pl.BlockSpec(memory_space=pltpu.SMEM),
            pl.BlockSpec((1, _GB, NPG), lambda i: (i, 0, 0)),
        ],
        out_specs=[
            pl.BlockSpec((_RB, D), lambda i: (i, 0)),
            pl.BlockSpec((1, _GB, NPG), lambda i: (i, 0, 0)),
            pl.BlockSpec((1, _GB, 2 * D), lambda i: (i, 0, 0)),
        ],
        out_shape=[
            jax.ShapeDtypeStruct((N, D), jnp.float32),
            jax.ShapeDtypeStruct((B // _GB, _GB, NPG), jnp.int32),
            jax.ShapeDtypeStruct((B // _GB, _GB, 2 * D), jnp.float32),
        ],
    )(part, hprev, wrel, brel, wroot, q, pn, og)


def _tcf_body(part_ref, hp_ref, wrel_ref, brel_ref, wroot_ref, q_ref,
              pn_ref, og_ref, r1_ref, r2_ref,
              w1_ref, b1_ref, w2_ref, b2_ref, w3_ref, b3_ref, out_ref, *, k):
    h, score = _block_conv(part_ref, hp_ref, wrel_ref, brel_ref, wroot_ref,
                           q_ref, pn_ref)
    for g in range(_GB):
        _, _, _, mx, mn = _graph_pool(h, score, og_ref, g, k)
        rg = pl.ds(g, 1)
        z = (r1_ref[0, rg, :] + r2_ref[0, rg, :]
             + jnp.concatenate([mx[None, :], mn[None, :]], axis=1))
        z = jnp.maximum(jnp.dot(z, w1_ref[...],
                                preferred_element_type=jnp.float32)
                        + b1_ref[...], 0.0)
        z = jnp.maximum(jnp.dot(z, w2_ref[...],
                                preferred_element_type=jnp.float32)
                        + b2_ref[...], 0.0)
        logits = jnp.dot(z, w3_ref[...],
                         preferred_element_type=jnp.float32) + b3_ref[...]
        mxl = jnp.max(logits, axis=1, keepdims=True)
        lse = jnp.log(jnp.sum(jnp.exp(logits - mxl), axis=1, keepdims=True))
        out_ref[0, rg, :] = logits - mxl - lse


def _tcf(part, hprev, wrel, brel, wroot, q, pn, og, r1, r2,
         w1, b1, w2, b2, w3, b3, k):
    return pl.pallas_call(
        functools.partial(_tcf_body, k=k),
        grid=(N // _RB,),
        in_specs=[
            pl.BlockSpec((_NC, _RB, D), lambda i: (0, i, 0)),
            pl.BlockSpec((_RB, D), lambda i: (i, 0)),
            pl.BlockSpec((D, D), lambda i: (0, 0)),
            pl.BlockSpec((1, D), lambda i: (0, 0)),
            pl.BlockSpec((D, D), lambda i: (0, 0)),
            pl.BlockSpec((D, 1), lambda i: (0, 0)),
            pl.BlockSpec(memory_space=pltpu.SMEM),
            pl.BlockSpec((1, _GB, NPG), lambda i: (i, 0, 0)),
            pl.BlockSpec((1, _GB, 2 * D), lambda i: (i, 0, 0)),
            pl.BlockSpec((1, _GB, 2 * D), lambda i: (i, 0, 0)),
            pl.BlockSpec((2 * D, D), lambda i: (0, 0)),
            pl.BlockSpec((1, D), lambda i: (0, 0)),
            pl.BlockSpec((D, 64), lambda i: (0, 0)),
            pl.BlockSpec((1, 64), lambda i: (0, 0)),
            pl.BlockSpec((64, 10), lambda i: (0, 0)),
            pl.BlockSpec((1, 10), lambda i: (0, 0)),
        ],
        out_specs=pl.BlockSpec((1, _GB, 10), lambda i: (i, 0, 0)),
        out_shape=jax.ShapeDtypeStruct((B // _GB, _GB, 10), jnp.float32),
    )(part, hprev, wrel, brel, wroot, q, pn, og, r1, r2,
      w1, b1, w2, b2, w3, b3)


def kernel(x, edge_index, batch, conv1_Wrel, conv1_brel, conv1_Wroot, p1,
           conv2_Wrel, conv2_brel, conv2_Wroot, p2,
           conv3_Wrel, conv3_brel, conv3_Wroot, p3,
           lin1_W, lin1_b, lin2_W, lin2_b, lin3_W, lin3_b):
    src = edge_index[0]
    dst = edge_index[1]
    zeros = jnp.zeros((_CH, D), jnp.float32)
    ord0 = jnp.tile(jnp.arange(NPG, dtype=jnp.int32), B).reshape(
        B // _GB, _GB, NPG)

    def pnorm(p):
        return (jnp.linalg.norm(p) + 1e-16).reshape(1, 1)

    part = _sc_segsum(x, src, dst, zeros)
    hm, ord1, r1 = _tcs(part, x, conv1_Wrel, conv1_brel.reshape(1, D),
                        conv1_Wroot, p1.reshape(D, 1), pnorm(p1), ord0, K1)

    part = _sc_segsum(hm, src, dst, zeros)
    hm, ord2, r2 = _tcs(part, hm, conv2_Wrel, conv2_brel.reshape(1, D),
                        conv2_Wroot, p2.reshape(D, 1), pnorm(p2), ord1, K2)

    part = _sc_segsum(hm, src, dst, zeros)
    out3 = _tcf(part, hm, conv3_Wrel, conv3_brel.reshape(1, D), conv3_Wroot,
                p3.reshape(D, 1), pnorm(p3), ord2, r1, r2,
                lin1_W, lin1_b.reshape(1, D), lin2_W, lin2_b.reshape(1, 64),
                lin3_W, lin3_b.reshape(1, 10), K3)
    return out3.reshape(B, 10)
